# Initial kernel scaffold; baseline (speedup 1.0000x reference)
import jax
import jax.numpy as jnp
from jax.experimental import pallas as pl


def kernel(x, Wr, br, W1, b1, W2, b2):
    raise NotImplementedError("write your pallas kernel here")


# speedup vs baseline: 4.8116x; 4.8116x over previous
import functools

import jax
import jax.numpy as jnp
from jax import lax
from jax.experimental import pallas as pl
from jax.experimental.pallas import tpu as pltpu
from jax.experimental.pallas import tpu_sc as plsc

B = 1
T = 2048
D = 1024
DFF = 2048
E = 64
CAP = int(T / E * 1.25)
NSLOT = E * CAP
YROWS = NSLOT + CAP
ALPHA = 0.01

NC = 2
NS = 16
NW = NC * NS
SPW = NSLOT // NW
TPW = T // NW

_CH = 256


def _router_body(x_ref, wr_ref, br_ref, slot_ref, gate_ref, counts_ref,
                 scal_ref):
    xf = x_ref[...]
    wr = wr_ref[...]
    logits = lax.dot_general(xf, wr, (((1,), (1,)), ((), ())),
                             preferred_element_type=jnp.float32)
    logits = logits + br_ref[...]
    lmax = jnp.max(logits, axis=1, keepdims=True)
    un = jnp.exp(logits - lmax)
    den = jnp.sum(un, axis=1, keepdims=True)
    probs = un / den
    pmax = jnp.max(probs, axis=1, keepdims=True)
    eio = lax.broadcasted_iota(jnp.int32, (T, E), 1)
    top1 = jnp.min(jnp.where(probs == pmax, eio, E), axis=1, keepdims=True)
    oh = (eio == top1).astype(jnp.float32)
    counts = jnp.sum(oh, axis=0, keepdims=True)

    rio = lax.broadcasted_iota(jnp.int32, (_CH, _CH), 0)
    cio = lax.broadcasted_iota(jnp.int32, (_CH, _CH), 1)
    tri = (rio >= cio).astype(jnp.float32)
    run = jnp.zeros((1, E), jnp.float32)
    chunks = []
    for i in range(T // _CH):
        ohc = oh[i * _CH:(i + 1) * _CH, :]
        csc = lax.dot_general(tri, ohc, (((1,), (0,)), ((), ())),
                              preferred_element_type=jnp.float32) + run
        run = run + jnp.sum(ohc, axis=0, keepdims=True)
        chunks.append(csc)
    cs = jnp.concatenate(chunks, axis=0)
    pos = jnp.sum(cs * oh, axis=1, keepdims=True).astype(jnp.int32) - 1
    keep = pos < CAP
    slot_ref[...] = jnp.where(keep, top1 * CAP + pos, NSLOT)
    gate_ref[...] = pmax
    counts_ref[...] = counts

    pcol = jnp.sum(probs, axis=0, keepdims=True)
    aux = (ALPHA * E) * jnp.sum((counts / T) * (pcol / T))
    dropped = jnp.sum(jnp.maximum(counts - float(CAP), 0.0))
    routed = jnp.maximum(jnp.sum(counts), 1.0)
    drop = dropped / routed
    li = lax.broadcasted_iota(jnp.int32, (1, 128), 1)
    scal_ref[...] = (jnp.where(li == 0, aux, 0.0)
                     + jnp.where(li == 1, drop, 0.0))


_router = pl.pallas_call(
    _router_body,
    out_shape=(
        jax.ShapeDtypeStruct((T, 1), jnp.int32),
        jax.ShapeDtypeStruct((T, 1), jnp.float32),
        jax.ShapeDtypeStruct((1, E), jnp.float32),
        jax.ShapeDtypeStruct((1, 128), jnp.float32),
    ),
)


def _dispatch_body(slot_hbm, gate_hbm, x_hbm, xg_hbm, gsl_hbm,
                   slot_v, gate_v, stok_v, idx_v, gs_v, rows_v, sem):
    wid = lax.axis_index("s") * NC + lax.axis_index("c")
    pltpu.sync_copy(slot_hbm, slot_v)
    pltpu.sync_copy(gate_hbm, gate_v)

    def init(i, c):
        stok_v[pl.ds(i * 16, 16)] = jnp.zeros((16,), jnp.int32)
        return c
    lax.fori_loop(0, NSLOT // 16, init, 0)

    def scat(i, c):
        s16 = slot_v[pl.ds(i * 16, 16)]
        t16 = lax.iota(jnp.int32, 16) + i * 16
        plsc.store_scatter(stok_v, [s16], t16, mask=s16 < NSLOT)
        return c
    lax.fori_loop(0, T // 16, scat, 0)

    base = wid * SPW

    def cp(i, c):
        c16 = stok_v[pl.ds(base + i * 16, 16)]
        idx_v[pl.ds(i * 16, 16)] = c16
        gs_v[pl.ds(i * 16, 16)] = plsc.load_gather(gate_v, [c16])
        return c
    lax.fori_loop(0, SPW // 16, cp, 0)

    pltpu.async_copy(x_hbm.at[idx_v], rows_v, sem).wait()
    pltpu.sync_copy(rows_v, xg_hbm.at[pl.ds(base, SPW)])
    pltpu.sync_copy(gs_v, gsl_hbm.at[pl.ds(base, SPW)])


def _ffn_body(xg_ref, gs_ref, w1_ref, b1_ref, w2_ref, b2_ref, yg_ref):
    e = pl.program_id(0)

    @pl.when(e < E)
    def _compute():
        xg = xg_ref[...]
        h = lax.dot_general(xg, w1_ref[0], (((1,), (1,)), ((), ())),
                            preferred_element_type=jnp.float32)
        h = jnp.maximum(h + b1_ref[0], 0.0)
        y = lax.dot_general(h, w2_ref[0], (((1,), (1,)), ((), ())),
                            preferred_element_type=jnp.float32)
        y = y + b2_ref[0]
        g = gs_ref[0]
        rio = lax.broadcasted_iota(jnp.int32, (CAP, CAP), 0)
        cio = lax.broadcasted_iota(jnp.int32, (CAP, CAP), 1)
        dg = jnp.where(rio == cio, jnp.broadcast_to(g, (CAP, CAP)), 0.0)
        gcol = jnp.sum(dg, axis=1, keepdims=True)
        yg_ref[...] = y * gcol

    @pl.when(e == E)
    def _zeros():
        yg_ref[...] = jnp.zeros((CAP, D), jnp.float32)


_ffn = pl.pallas_call(
    _ffn_body,
    grid=(E + 1,),
    in_specs=[
        pl.BlockSpec((CAP, D), lambda e: (jnp.minimum(e, E - 1), 0)),
        pl.BlockSpec((1, 1, CAP), lambda e: (jnp.minimum(e, E - 1), 0, 0)),
        pl.BlockSpec((1, DFF, D), lambda e: (jnp.minimum(e, E - 1), 0, 0)),
        pl.BlockSpec((1, 1, DFF), lambda e: (jnp.minimum(e, E - 1), 0, 0)),
        pl.BlockSpec((1, D, DFF), lambda e: (jnp.minimum(e, E - 1), 0, 0)),
        pl.BlockSpec((1, 1, D), lambda e: (jnp.minimum(e, E - 1), 0, 0)),
    ],
    out_specs=pl.BlockSpec((CAP, D), lambda e: (e, 0)),
    out_shape=jax.ShapeDtypeStruct((YROWS, D), jnp.float32),
)


def _combine_body(yg_hbm, slot_hbm, out_hbm, sl_v, rows_v, sem):
    wid = lax.axis_index("s") * NC + lax.axis_index("c")
    base = wid * TPW
    pltpu.sync_copy(slot_hbm.at[pl.ds(base, TPW)], sl_v)
    pltpu.async_copy(yg_hbm.at[sl_v], rows_v, sem).wait()
    pltpu.sync_copy(rows_v, out_hbm.at[pl.ds(base, TPW)])


@functools.lru_cache(maxsize=1)
def _sc_kernels():
    mesh = plsc.VectorSubcoreMesh(core_axis_name="c", subcore_axis_name="s",
                                  num_cores=NC, num_subcores=NS)
    sc_params = pltpu.CompilerParams(needs_layout_passes=False)
    dispatch = pl.kernel(
        _dispatch_body,
        out_type=(
            jax.ShapeDtypeStruct((NSLOT, D), jnp.float32),
            jax.ShapeDtypeStruct((NSLOT,), jnp.float32),
        ),
        mesh=mesh,
        scratch_types=[
            pltpu.VMEM((T,), jnp.int32),
            pltpu.VMEM((T,), jnp.float32),
            pltpu.VMEM((NSLOT,), jnp.int32),
            pltpu.VMEM((SPW,), jnp.int32),
            pltpu.VMEM((SPW,), jnp.float32),
            pltpu.VMEM((SPW, D), jnp.float32),
            pltpu.SemaphoreType.DMA,
        ],
        compiler_params=sc_params,
    )
    combine = pl.kernel(
        _combine_body,
        out_type=jax.ShapeDtypeStruct((T, D), jnp.float32),
        mesh=mesh,
        scratch_types=[
            pltpu.VMEM((TPW,), jnp.int32),
            pltpu.VMEM((TPW, D), jnp.float32),
            pltpu.SemaphoreType.DMA,
        ],
        compiler_params=sc_params,
    )
    return dispatch, combine


def kernel(x, Wr, br, W1, b1, W2, b2):
    _dispatch, _combine = _sc_kernels()
    xf = x.reshape(T, D)
    slot2, gate2, counts2, scal = _router(xf, Wr, br.reshape(1, E))
    slot = slot2.reshape(T)
    gate = gate2.reshape(T)
    xg, gsl = _dispatch(slot, gate, xf)
    yg = _ffn(xg, gsl.reshape(E, 1, CAP), W1, b1.reshape(E, 1, DFF),
              W2, b2.reshape(E, 1, D))
    out_flat = _combine(yg, slot)
    out = out_flat.reshape(x.shape)
    counts = counts2.reshape(E).astype(jnp.int32)
    return out, scal[0, 0], scal[0, 1], counts


# speedup vs baseline: 4.8260x; 1.0030x over previous
import functools

import jax
import jax.numpy as jnp
from jax import lax
from jax.experimental import pallas as pl
from jax.experimental.pallas import tpu as pltpu
from jax.experimental.pallas import tpu_sc as plsc

B = 1
T = 2048
D = 1024
DFF = 2048
E = 64
CAP = int(T / E * 1.25)
NSLOT = E * CAP
YROWS = NSLOT + CAP
ALPHA = 0.01

NC = 2
NS = 16
NW = NC * NS
SPW = NSLOT // NW
TPW = T // NW

_CH = 256


def _router_body(x_ref, wr_ref, br_ref, slot_ref, gate_ref, counts_ref,
                 scal_ref):
    xf = x_ref[...]
    wr = wr_ref[...]
    logits = lax.dot_general(xf, wr, (((1,), (1,)), ((), ())),
                             preferred_element_type=jnp.float32)
    logits = logits + br_ref[...]
    lmax = jnp.max(logits, axis=1, keepdims=True)
    un = jnp.exp(logits - lmax)
    den = jnp.sum(un, axis=1, keepdims=True)
    probs = un / den
    pmax = jnp.max(probs, axis=1, keepdims=True)
    eio = lax.broadcasted_iota(jnp.int32, (T, E), 1)
    top1 = jnp.min(jnp.where(probs == pmax, eio, E), axis=1, keepdims=True)
    oh = (eio == top1).astype(jnp.float32)
    counts = jnp.sum(oh, axis=0, keepdims=True)

    rio = lax.broadcasted_iota(jnp.int32, (_CH, _CH), 0)
    cio = lax.broadcasted_iota(jnp.int32, (_CH, _CH), 1)
    tri = (rio >= cio).astype(jnp.float32)
    run = jnp.zeros((1, E), jnp.float32)
    chunks = []
    for i in range(T // _CH):
        ohc = oh[i * _CH:(i + 1) * _CH, :]
        csc = lax.dot_general(tri, ohc, (((1,), (0,)), ((), ())),
                              preferred_element_type=jnp.float32) + run
        run = run + jnp.sum(ohc, axis=0, keepdims=True)
        chunks.append(csc)
    cs = jnp.concatenate(chunks, axis=0)
    pos = jnp.sum(cs * oh, axis=1, keepdims=True).astype(jnp.int32) - 1
    keep = pos < CAP
    slot_ref[...] = jnp.where(keep, top1 * CAP + pos, NSLOT)
    gate_ref[...] = pmax
    counts_ref[...] = counts

    pcol = jnp.sum(probs, axis=0, keepdims=True)
    aux = (ALPHA * E) * jnp.sum((counts / T) * (pcol / T))
    dropped = jnp.sum(jnp.maximum(counts - float(CAP), 0.0))
    routed = jnp.maximum(jnp.sum(counts), 1.0)
    drop = dropped / routed
    li = lax.broadcasted_iota(jnp.int32, (1, 128), 1)
    scal_ref[...] = (jnp.where(li == 0, aux, 0.0)
                     + jnp.where(li == 1, drop, 0.0))


_router = pl.pallas_call(
    _router_body,
    out_shape=(
        jax.ShapeDtypeStruct((T, 1), jnp.int32),
        jax.ShapeDtypeStruct((T, 1), jnp.float32),
        jax.ShapeDtypeStruct((1, E), jnp.float32),
        jax.ShapeDtypeStruct((1, 128), jnp.float32),
    ),
)


def _dispatch_body(slot_hbm, gate_hbm, x_hbm, xg_hbm, gsl_hbm,
                   slot_v, gate_v, win_v, gs_v, rows_v, sem):
    wid = lax.axis_index("s") * NC + lax.axis_index("c")
    base = wid * SPW
    pltpu.sync_copy(slot_hbm, slot_v)
    pltpu.sync_copy(gate_hbm, gate_v)

    @plsc.parallel_loop(0, SPW, step=16)
    def _init(i):
        win_v[pl.ds(i, 16)] = jnp.zeros((16,), jnp.int32)

    @plsc.parallel_loop(0, T, step=16, unroll=8)
    def _scat(i):
        s16 = slot_v[pl.ds(i, 16)]
        t16 = lax.iota(jnp.int32, 16) + i
        w16 = s16 - base
        m = (w16 >= 0) & (w16 < SPW)
        plsc.store_scatter(win_v, [jnp.where(m, w16, 0)], t16, mask=m)

    @plsc.parallel_loop(0, SPW, step=16)
    def _gate(i):
        c16 = win_v[pl.ds(i, 16)]
        gs_v[pl.ds(i, 16)] = plsc.load_gather(gate_v, [c16])

    pltpu.async_copy(x_hbm.at[win_v], rows_v, sem).wait()
    pltpu.sync_copy(rows_v, xg_hbm.at[pl.ds(base, SPW)])
    pltpu.sync_copy(gs_v, gsl_hbm.at[pl.ds(base, SPW)])


def _ffn_body(xg_ref, gs_ref, w1_ref, b1_ref, w2_ref, b2_ref, yg_ref):
    e = pl.program_id(0)

    @pl.when(e < E)
    def _compute():
        xg = xg_ref[...]
        h = lax.dot_general(xg, w1_ref[0], (((1,), (1,)), ((), ())),
                            preferred_element_type=jnp.float32)
        h = jnp.maximum(h + b1_ref[0], 0.0)
        y = lax.dot_general(h, w2_ref[0], (((1,), (1,)), ((), ())),
                            preferred_element_type=jnp.float32)
        y = y + b2_ref[0]
        g = gs_ref[0]
        rio = lax.broadcasted_iota(jnp.int32, (CAP, CAP), 0)
        cio = lax.broadcasted_iota(jnp.int32, (CAP, CAP), 1)
        dg = jnp.where(rio == cio, jnp.broadcast_to(g, (CAP, CAP)), 0.0)
        gcol = jnp.sum(dg, axis=1, keepdims=True)
        yg_ref[...] = y * gcol

    @pl.when(e == E)
    def _zeros():
        yg_ref[...] = jnp.zeros((CAP, D), jnp.float32)


_ffn = pl.pallas_call(
    _ffn_body,
    grid=(E + 1,),
    in_specs=[
        pl.BlockSpec((CAP, D), lambda e: (jnp.minimum(e, E - 1), 0)),
        pl.BlockSpec((1, 1, CAP), lambda e: (jnp.minimum(e, E - 1), 0, 0)),
        pl.BlockSpec((1, DFF, D), lambda e: (jnp.minimum(e, E - 1), 0, 0)),
        pl.BlockSpec((1, 1, DFF), lambda e: (jnp.minimum(e, E - 1), 0, 0)),
        pl.BlockSpec((1, D, DFF), lambda e: (jnp.minimum(e, E - 1), 0, 0)),
        pl.BlockSpec((1, 1, D), lambda e: (jnp.minimum(e, E - 1), 0, 0)),
    ],
    out_specs=pl.BlockSpec((CAP, D), lambda e: (e, 0)),
    out_shape=jax.ShapeDtypeStruct((YROWS, D), jnp.float32),
)


def _combine_body(yg_hbm, slot_hbm, out_hbm, sl_v, rows_v, sem):
    wid = lax.axis_index("s") * NC + lax.axis_index("c")
    base = wid * TPW
    pltpu.sync_copy(slot_hbm.at[pl.ds(base, TPW)], sl_v)
    pltpu.async_copy(yg_hbm.at[sl_v], rows_v, sem).wait()
    pltpu.sync_copy(rows_v, out_hbm.at[pl.ds(base, TPW)])


@functools.lru_cache(maxsize=1)
def _sc_kernels():
    mesh = plsc.VectorSubcoreMesh(core_axis_name="c", subcore_axis_name="s",
                                  num_cores=NC, num_subcores=NS)
    sc_params = pltpu.CompilerParams(needs_layout_passes=False)
    dispatch = pl.kernel(
        _dispatch_body,
        out_type=(
            jax.ShapeDtypeStruct((NSLOT, D), jnp.float32),
            jax.ShapeDtypeStruct((NSLOT,), jnp.float32),
        ),
        mesh=mesh,
        scratch_types=[
            pltpu.VMEM((T,), jnp.int32),
            pltpu.VMEM((T,), jnp.float32),
            pltpu.VMEM((SPW,), jnp.int32),
            pltpu.VMEM((SPW,), jnp.float32),
            pltpu.VMEM((SPW, D), jnp.float32),
            pltpu.SemaphoreType.DMA,
        ],
        compiler_params=sc_params,
    )
    combine = pl.kernel(
        _combine_body,
        out_type=jax.ShapeDtypeStruct((T, D), jnp.float32),
        mesh=mesh,
        scratch_types=[
            pltpu.VMEM((TPW,), jnp.int32),
            pltpu.VMEM((TPW, D), jnp.float32),
            pltpu.SemaphoreType.DMA,
        ],
        compiler_params=sc_params,
    )
    return dispatch, combine


def kernel(x, Wr, br, W1, b1, W2, b2):
    _dispatch, _combine = _sc_kernels()
    xf = x.reshape(T, D)
    slot2, gate2, counts2, scal = _router(xf, Wr, br.reshape(1, E))
    slot = slot2.reshape(T)
    gate = gate2.reshape(T)
    xg, gsl = _dispatch(slot, gate, xf)
    yg = _ffn(xg, gsl.reshape(E, 1, CAP), W1, b1.reshape(E, 1, DFF),
              W2, b2.reshape(E, 1, D))
    out_flat = _combine(yg, slot)
    out = out_flat.reshape(x.shape)
    counts = counts2.reshape(E).astype(jnp.int32)
    return out, scal[0, 0], scal[0, 1], counts


# speedup vs baseline: 5.2684x; 1.0917x over previous
import functools

import jax
import jax.numpy as jnp
from jax import lax
from jax.experimental import pallas as pl
from jax.experimental.pallas import tpu as pltpu
from jax.experimental.pallas import tpu_sc as plsc

B = 1
T = 2048
D = 1024
DFF = 2048
E = 64
CAP = int(T / E * 1.25)
NSLOT = E * CAP
YROWS = NSLOT + CAP
ALPHA = 0.01

NC = 2
NS = 16
NW = NC * NS
SPW = NSLOT // NW
TPW = T // NW

_CH = 256


def _router_body(x_ref, wr_ref, br_ref, slot_ref, gate_ref, counts_ref,
                 scal_ref):
    xf = x_ref[...]
    wr = wr_ref[...]
    logits = lax.dot_general(xf, wr, (((1,), (1,)), ((), ())),
                             preferred_element_type=jnp.float32)
    logits = logits + br_ref[...]
    lmax = jnp.max(logits, axis=1, keepdims=True)
    un = jnp.exp(logits - lmax)
    den = jnp.sum(un, axis=1, keepdims=True)
    probs = un / den
    pmax = jnp.max(probs, axis=1, keepdims=True)
    eio = lax.broadcasted_iota(jnp.int32, (T, E), 1)
    top1 = jnp.min(jnp.where(probs == pmax, eio, E), axis=1, keepdims=True)
    oh = (eio == top1).astype(jnp.float32)
    counts = jnp.sum(oh, axis=0, keepdims=True)

    rio = lax.broadcasted_iota(jnp.int32, (_CH, _CH), 0)
    cio = lax.broadcasted_iota(jnp.int32, (_CH, _CH), 1)
    tri = (rio >= cio).astype(jnp.float32)
    run = jnp.zeros((1, E), jnp.float32)
    chunks = []
    for i in range(T // _CH):
        ohc = oh[i * _CH:(i + 1) * _CH, :]
        csc = lax.dot_general(tri, ohc, (((1,), (0,)), ((), ())),
                              preferred_element_type=jnp.float32) + run
        run = run + jnp.sum(ohc, axis=0, keepdims=True)
        chunks.append(csc)
    cs = jnp.concatenate(chunks, axis=0)
    pos = jnp.sum(cs * oh, axis=1, keepdims=True).astype(jnp.int32) - 1
    keep = pos < CAP
    slot_ref[...] = jnp.where(keep, top1 * CAP + pos, NSLOT)
    gate_ref[...] = pmax
    counts_ref[...] = counts

    pcol = jnp.sum(probs, axis=0, keepdims=True)
    aux = (ALPHA * E) * jnp.sum((counts / T) * (pcol / T))
    dropped = jnp.sum(jnp.maximum(counts - float(CAP), 0.0))
    routed = jnp.maximum(jnp.sum(counts), 1.0)
    drop = dropped / routed
    li = lax.broadcasted_iota(jnp.int32, (1, 128), 1)
    scal_ref[...] = (jnp.where(li == 0, aux, 0.0)
                     + jnp.where(li == 1, drop, 0.0))


_router = pl.pallas_call(
    _router_body,
    out_shape=(
        jax.ShapeDtypeStruct((T, 1), jnp.int32),
        jax.ShapeDtypeStruct((T, 1), jnp.float32),
        jax.ShapeDtypeStruct((1, E), jnp.float32),
        jax.ShapeDtypeStruct((1, 128), jnp.float32),
    ),
)


def _ffn_body(slot_ref, gate_ref, x_ref, w1_ref, b1_ref, w2_ref, b2_ref,
              yg_ref):
    e = pl.program_id(0)

    @pl.when(e < E)
    def _compute():
        sl = slot_ref[...]
        cio = lax.broadcasted_iota(jnp.int32, (T, CAP), 1) + e * CAP
        onehot = (sl == cio).astype(jnp.float32)
        xg = lax.dot_general(onehot, x_ref[...], (((0,), (0,)), ((), ())),
                             preferred_element_type=jnp.float32)
        h = lax.dot_general(xg, w1_ref[0], (((1,), (1,)), ((), ())),
                            preferred_element_type=jnp.float32)
        h = jnp.maximum(h + b1_ref[0], 0.0)
        y = lax.dot_general(h, w2_ref[0], (((1,), (1,)), ((), ())),
                            preferred_element_type=jnp.float32)
        y = y + b2_ref[0]
        grow = jnp.sum(onehot * gate_ref[...], axis=0, keepdims=True)
        rio = lax.broadcasted_iota(jnp.int32, (CAP, CAP), 0)
        dio = lax.broadcasted_iota(jnp.int32, (CAP, CAP), 1)
        dg = jnp.where(rio == dio, jnp.broadcast_to(grow, (CAP, CAP)), 0.0)
        gcol = jnp.sum(dg, axis=1, keepdims=True)
        yg_ref[...] = y * gcol

    @pl.when(e == E)
    def _zeros():
        yg_ref[...] = jnp.zeros((CAP, D), jnp.float32)


_ffn = pl.pallas_call(
    _ffn_body,
    grid=(E + 1,),
    in_specs=[
        pl.BlockSpec((T, 1), lambda e: (0, 0)),
        pl.BlockSpec((T, 1), lambda e: (0, 0)),
        pl.BlockSpec((T, D), lambda e: (0, 0)),
        pl.BlockSpec((1, DFF, D), lambda e: (jnp.minimum(e, E - 1), 0, 0)),
        pl.BlockSpec((1, 1, DFF), lambda e: (jnp.minimum(e, E - 1), 0, 0)),
        pl.BlockSpec((1, D, DFF), lambda e: (jnp.minimum(e, E - 1), 0, 0)),
        pl.BlockSpec((1, 1, D), lambda e: (jnp.minimum(e, E - 1), 0, 0)),
    ],
    out_specs=pl.BlockSpec((CAP, D), lambda e: (e, 0)),
    out_shape=jax.ShapeDtypeStruct((YROWS, D), jnp.float32),
)


def _combine_body(yg_hbm, slot_hbm, out_hbm, sl_v, rows_v, sem):
    wid = lax.axis_index("s") * NC + lax.axis_index("c")
    base = wid * TPW
    pltpu.sync_copy(slot_hbm.at[pl.ds(base, TPW)], sl_v)
    pltpu.async_copy(yg_hbm.at[sl_v], rows_v, sem).wait()
    pltpu.sync_copy(rows_v, out_hbm.at[pl.ds(base, TPW)])


@functools.lru_cache(maxsize=1)
def _sc_kernels():
    mesh = plsc.VectorSubcoreMesh(core_axis_name="c", subcore_axis_name="s",
                                  num_cores=NC, num_subcores=NS)
    sc_params = pltpu.CompilerParams(needs_layout_passes=False)
    combine = pl.kernel(
        _combine_body,
        out_type=jax.ShapeDtypeStruct((T, D), jnp.float32),
        mesh=mesh,
        scratch_types=[
            pltpu.VMEM((TPW,), jnp.int32),
            pltpu.VMEM((TPW, D), jnp.float32),
            pltpu.SemaphoreType.DMA,
        ],
        compiler_params=sc_params,
    )
    return combine


def kernel(x, Wr, br, W1, b1, W2, b2):
    _combine = _sc_kernels()
    xf = x.reshape(T, D)
    slot2, gate2, counts2, scal = _router(xf, Wr, br.reshape(1, E))
    yg = _ffn(slot2, gate2, xf, W1, b1.reshape(E, 1, DFF),
              W2, b2.reshape(E, 1, D))
    out_flat = _combine(yg, slot2.reshape(T))
    out = out_flat.reshape(x.shape)
    counts = counts2.reshape(E).astype(jnp.int32)
    return out, scal[0, 0], scal[0, 1], counts


# speedup vs baseline: 5.3422x; 1.0140x over previous
import functools

import jax
import jax.numpy as jnp
from jax import lax
from jax.experimental import pallas as pl
from jax.experimental.pallas import tpu as pltpu
from jax.experimental.pallas import tpu_sc as plsc

B = 1
T = 2048
D = 1024
DFF = 2048
E = 64
CAP = int(T / E * 1.25)
NSLOT = E * CAP
YROWS = NSLOT + CAP
ALPHA = 0.01

NC = 2
NS = 16
NW = NC * NS
SPW = NSLOT // NW
TPW = T // NW

_CH = 256


def _route_compute(xf, wr, br_row):
    logits = lax.dot_general(xf, wr, (((1,), (1,)), ((), ())),
                             preferred_element_type=jnp.float32)
    logits = logits + br_row
    lmax = jnp.max(logits, axis=1, keepdims=True)
    un = jnp.exp(logits - lmax)
    den = jnp.sum(un, axis=1, keepdims=True)
    probs = un / den
    pmax = jnp.max(probs, axis=1, keepdims=True)
    eio = lax.broadcasted_iota(jnp.int32, (T, E), 1)
    top1 = jnp.min(jnp.where(probs == pmax, eio, E), axis=1, keepdims=True)
    oh = (eio == top1).astype(jnp.float32)
    counts = jnp.sum(oh, axis=0, keepdims=True)

    rio = lax.broadcasted_iota(jnp.int32, (_CH, _CH), 0)
    cio = lax.broadcasted_iota(jnp.int32, (_CH, _CH), 1)
    tri = (rio >= cio).astype(jnp.float32)
    run = jnp.zeros((1, E), jnp.float32)
    chunks = []
    for i in range(T // _CH):
        ohc = oh[i * _CH:(i + 1) * _CH, :]
        csc = lax.dot_general(tri, ohc, (((1,), (0,)), ((), ())),
                              preferred_element_type=jnp.float32) + run
        run = run + jnp.sum(ohc, axis=0, keepdims=True)
        chunks.append(csc)
    cs = jnp.concatenate(chunks, axis=0)
    pos = jnp.sum(cs * oh, axis=1, keepdims=True).astype(jnp.int32) - 1
    keep = pos < CAP
    slot = jnp.where(keep, top1 * CAP + pos, NSLOT)

    pcol = jnp.sum(probs, axis=0, keepdims=True)
    aux = (ALPHA * E) * jnp.sum((counts / T) * (pcol / T))
    dropped = jnp.sum(jnp.maximum(counts - float(CAP), 0.0))
    routed = jnp.maximum(jnp.sum(counts), 1.0)
    drop = dropped / routed
    li = lax.broadcasted_iota(jnp.int32, (1, 128), 1)
    scal = (jnp.where(li == 0, aux, 0.0)
            + jnp.where(li == 1, drop, 0.0))
    return slot, pmax, counts, scal


def _ffn_body(x_ref, wr_ref, br_ref, w1_ref, b1_ref, w2_ref, b2_ref,
              yg_ref, slot_out_ref, counts_ref, scal_ref,
              slot_s, gate_s):
    e = pl.program_id(0)

    @pl.when(e == 0)
    def _route():
        slot, gate, counts, scal = _route_compute(x_ref[...], wr_ref[...],
                                                  br_ref[...])
        slot_s[...] = slot
        gate_s[...] = gate
        slot_out_ref[...] = slot
        counts_ref[...] = counts
        scal_ref[...] = scal

    @pl.when(e < E)
    def _compute():
        sl = slot_s[...]
        cio = lax.broadcasted_iota(jnp.int32, (T, CAP), 1) + e * CAP
        onehot = (sl == cio).astype(jnp.float32)
        xg = lax.dot_general(onehot, x_ref[...], (((0,), (0,)), ((), ())),
                             preferred_element_type=jnp.float32)
        h = lax.dot_general(xg, w1_ref[0], (((1,), (1,)), ((), ())),
                            preferred_element_type=jnp.float32)
        h = jnp.maximum(h + b1_ref[0], 0.0)
        y = lax.dot_general(h, w2_ref[0], (((1,), (1,)), ((), ())),
                            preferred_element_type=jnp.float32)
        y = y + b2_ref[0]
        grow = jnp.sum(onehot * gate_s[...], axis=0, keepdims=True)
        rio = lax.broadcasted_iota(jnp.int32, (CAP, CAP), 0)
        dio = lax.broadcasted_iota(jnp.int32, (CAP, CAP), 1)
        dg = jnp.where(rio == dio, jnp.broadcast_to(grow, (CAP, CAP)), 0.0)
        gcol = jnp.sum(dg, axis=1, keepdims=True)
        yg_ref[...] = y * gcol

    @pl.when(e == E)
    def _zeros():
        yg_ref[...] = jnp.zeros((CAP, D), jnp.float32)


_ffn = pl.pallas_call(
    _ffn_body,
    grid=(E + 1,),
    in_specs=[
        pl.BlockSpec((T, D), lambda e: (0, 0)),
        pl.BlockSpec((E, D), lambda e: (0, 0)),
        pl.BlockSpec((1, E), lambda e: (0, 0)),
        pl.BlockSpec((1, DFF, D), lambda e: (jnp.minimum(e, E - 1), 0, 0)),
        pl.BlockSpec((1, 1, DFF), lambda e: (jnp.minimum(e, E - 1), 0, 0)),
        pl.BlockSpec((1, D, DFF), lambda e: (jnp.minimum(e, E - 1), 0, 0)),
        pl.BlockSpec((1, 1, D), lambda e: (jnp.minimum(e, E - 1), 0, 0)),
    ],
    out_specs=(
        pl.BlockSpec((CAP, D), lambda e: (e, 0)),
        pl.BlockSpec((T, 1), lambda e: (0, 0)),
        pl.BlockSpec((1, E), lambda e: (0, 0)),
        pl.BlockSpec((1, 128), lambda e: (0, 0)),
    ),
    out_shape=(
        jax.ShapeDtypeStruct((YROWS, D), jnp.float32),
        jax.ShapeDtypeStruct((T, 1), jnp.int32),
        jax.ShapeDtypeStruct((1, E), jnp.float32),
        jax.ShapeDtypeStruct((1, 128), jnp.float32),
    ),
    scratch_shapes=[
        pltpu.VMEM((T, 1), jnp.int32),
        pltpu.VMEM((T, 1), jnp.float32),
    ],
)


def _combine_body(yg_hbm, slot_hbm, out_hbm, sl_v, rows_v, sem):
    wid = lax.axis_index("s") * NC + lax.axis_index("c")
    base = wid * TPW
    pltpu.sync_copy(slot_hbm.at[pl.ds(base, TPW)], sl_v)
    pltpu.async_copy(yg_hbm.at[sl_v], rows_v, sem).wait()
    pltpu.sync_copy(rows_v, out_hbm.at[pl.ds(base, TPW)])


@functools.lru_cache(maxsize=1)
def _sc_kernels():
    mesh = plsc.VectorSubcoreMesh(core_axis_name="c", subcore_axis_name="s",
                                  num_cores=NC, num_subcores=NS)
    sc_params = pltpu.CompilerParams(needs_layout_passes=False)
    combine = pl.kernel(
        _combine_body,
        out_type=jax.ShapeDtypeStruct((T, D), jnp.float32),
        mesh=mesh,
        scratch_types=[
            pltpu.VMEM((TPW,), jnp.int32),
            pltpu.VMEM((TPW, D), jnp.float32),
            pltpu.SemaphoreType.DMA,
        ],
        compiler_params=sc_params,
    )
    return combine


def kernel(x, Wr, br, W1, b1, W2, b2):
    _combine = _sc_kernels()
    xf = x.reshape(T, D)
    yg, slot2, counts2, scal = _ffn(xf, Wr, br.reshape(1, E), W1,
                                    b1.reshape(E, 1, DFF), W2,
                                    b2.reshape(E, 1, D))
    out_flat = _combine(yg, slot2.reshape(T))
    out = out_flat.reshape(x.shape)
    counts = counts2.reshape(E).astype(jnp.int32)
    return out, scal[0, 0], scal[0, 1], counts


# speedup vs baseline: 5.3708x; 1.0054x over previous
import functools

import jax
import jax.numpy as jnp
from jax import lax
from jax.experimental import pallas as pl
from jax.experimental.pallas import tpu as pltpu
from jax.experimental.pallas import tpu_sc as plsc

B = 1
T = 2048
D = 1024
DFF = 2048
E = 64
CAP = int(T / E * 1.25)
NSLOT = E * CAP
YROWS = NSLOT + CAP
ALPHA = 0.01

NC = 2
NS = 16
NW = NC * NS
SPW = NSLOT // NW
TPW = T // NW

_CH = 256


def _route_compute(xf, wr, br_row):
    logits = lax.dot_general(xf, wr, (((1,), (1,)), ((), ())),
                             preferred_element_type=jnp.float32)
    logits = logits + br_row
    lmax = jnp.max(logits, axis=1, keepdims=True)
    un = jnp.exp(logits - lmax)
    den = jnp.sum(un, axis=1, keepdims=True)
    probs = un / den
    pmax = jnp.max(probs, axis=1, keepdims=True)
    eio = lax.broadcasted_iota(jnp.int32, (T, E), 1)
    top1 = jnp.min(jnp.where(probs == pmax, eio, E), axis=1, keepdims=True)
    oh = (eio == top1).astype(jnp.float32)
    counts = jnp.sum(oh, axis=0, keepdims=True)

    rio = lax.broadcasted_iota(jnp.int32, (_CH, _CH), 0)
    cio = lax.broadcasted_iota(jnp.int32, (_CH, _CH), 1)
    tri = (rio >= cio).astype(jnp.float32)
    run = jnp.zeros((1, E), jnp.float32)
    chunks = []
    for i in range(T // _CH):
        ohc = oh[i * _CH:(i + 1) * _CH, :]
        csc = lax.dot_general(tri, ohc, (((1,), (0,)), ((), ())),
                              preferred_element_type=jnp.float32) + run
        run = run + jnp.sum(ohc, axis=0, keepdims=True)
        chunks.append(csc)
    cs = jnp.concatenate(chunks, axis=0)
    pos = jnp.sum(cs * oh, axis=1, keepdims=True).astype(jnp.int32) - 1
    keep = pos < CAP
    slot = jnp.where(keep, top1 * CAP + pos, NSLOT)

    pcol = jnp.sum(probs, axis=0, keepdims=True)
    aux = (ALPHA * E) * jnp.sum((counts / T) * (pcol / T))
    dropped = jnp.sum(jnp.maximum(counts - float(CAP), 0.0))
    routed = jnp.maximum(jnp.sum(counts), 1.0)
    drop = dropped / routed
    li = lax.broadcasted_iota(jnp.int32, (1, 128), 1)
    scal = (jnp.where(li == 0, aux, 0.0)
            + jnp.where(li == 1, drop, 0.0))
    return slot, pmax, counts, scal


def _ffn_body(x_ref, wr_ref, br_ref, w1a_ref, w1b_ref, b1_ref, w2a_ref,
              w2b_ref, b2_ref,
              yg_ref, slot_out_ref, counts_ref, scal_ref,
              slot_s, gate_s):
    e = pl.program_id(0)

    @pl.when(e == 0)
    def _route():
        slot, gate, counts, scal = _route_compute(x_ref[...], wr_ref[...],
                                                  br_ref[...])
        slot_s[...] = slot
        gate_s[...] = gate
        slot_out_ref[...] = slot
        counts_ref[...] = counts
        scal_ref[...] = scal

    @pl.when(e < E)
    def _compute():
        sl = slot_s[...]
        cio = lax.broadcasted_iota(jnp.int32, (T, CAP), 1) + e * CAP
        onehot = (sl == cio).astype(jnp.float32)
        xg = lax.dot_general(onehot, x_ref[...], (((0,), (0,)), ((), ())),
                             preferred_element_type=jnp.float32)
        h1 = lax.dot_general(xg, w1a_ref[0], (((1,), (1,)), ((), ())),
                             preferred_element_type=jnp.float32)
        h1 = jnp.maximum(h1 + b1_ref[0, :, :DFF // 2], 0.0)
        h2 = lax.dot_general(xg, w1b_ref[0], (((1,), (1,)), ((), ())),
                             preferred_element_type=jnp.float32)
        h2 = jnp.maximum(h2 + b1_ref[0, :, DFF // 2:], 0.0)
        grow = jnp.sum(onehot * gate_s[...], axis=0, keepdims=True)
        rio = lax.broadcasted_iota(jnp.int32, (CAP, CAP), 0)
        dio = lax.broadcasted_iota(jnp.int32, (CAP, CAP), 1)
        dg = jnp.where(rio == dio, jnp.broadcast_to(grow, (CAP, CAP)), 0.0)
        gcol = jnp.sum(dg, axis=1, keepdims=True)
        w2a = w2a_ref[0]
        w2b = w2b_ref[0]
        dn = (((1,), (1,)), ((), ()))
        ya = (lax.dot_general(h1, w2a[:, :DFF // 2], dn,
                              preferred_element_type=jnp.float32)
              + lax.dot_general(h2, w2a[:, DFF // 2:], dn,
                                preferred_element_type=jnp.float32))
        yb = (lax.dot_general(h1, w2b[:, :DFF // 2], dn,
                              preferred_element_type=jnp.float32)
              + lax.dot_general(h2, w2b[:, DFF // 2:], dn,
                                preferred_element_type=jnp.float32))
        yg_ref[:, :D // 2] = (ya + b2_ref[0, :, :D // 2]) * gcol
        yg_ref[:, D // 2:] = (yb + b2_ref[0, :, D // 2:]) * gcol

    @pl.when(e == E)
    def _zeros():
        yg_ref[...] = jnp.zeros((CAP, D), jnp.float32)


_ffn = pl.pallas_call(
    _ffn_body,
    grid=(E + 1,),
    in_specs=[
        pl.BlockSpec((T, D), lambda e: (0, 0)),
        pl.BlockSpec((E, D), lambda e: (0, 0)),
        pl.BlockSpec((1, E), lambda e: (0, 0)),
        pl.BlockSpec((1, DFF // 2, D), lambda e: (jnp.minimum(e, E - 1), 0, 0)),
        pl.BlockSpec((1, DFF // 2, D), lambda e: (jnp.minimum(e, E - 1), 1, 0)),
        pl.BlockSpec((1, 1, DFF), lambda e: (jnp.minimum(e, E - 1), 0, 0)),
        pl.BlockSpec((1, D // 2, DFF), lambda e: (jnp.minimum(e, E - 1), 0, 0)),
        pl.BlockSpec((1, D // 2, DFF), lambda e: (jnp.minimum(e, E - 1), 1, 0)),
        pl.BlockSpec((1, 1, D), lambda e: (jnp.minimum(e, E - 1), 0, 0)),
    ],
    out_specs=(
        pl.BlockSpec((CAP, D), lambda e: (e, 0)),
        pl.BlockSpec((T, 1), lambda e: (0, 0)),
        pl.BlockSpec((1, E), lambda e: (0, 0)),
        pl.BlockSpec((1, 128), lambda e: (0, 0)),
    ),
    out_shape=(
        jax.ShapeDtypeStruct((YROWS, D), jnp.float32),
        jax.ShapeDtypeStruct((T, 1), jnp.int32),
        jax.ShapeDtypeStruct((1, E), jnp.float32),
        jax.ShapeDtypeStruct((1, 128), jnp.float32),
    ),
    scratch_shapes=[
        pltpu.VMEM((T, 1), jnp.int32),
        pltpu.VMEM((T, 1), jnp.float32),
    ],
)


def _combine_body(yg_hbm, slot_hbm, out_hbm, sl_v, rows_v, sem):
    wid = lax.axis_index("s") * NC + lax.axis_index("c")
    base = wid * TPW
    pltpu.sync_copy(slot_hbm.at[pl.ds(base, TPW)], sl_v)
    pltpu.async_copy(yg_hbm.at[sl_v], rows_v, sem).wait()
    pltpu.sync_copy(rows_v, out_hbm.at[pl.ds(base, TPW)])


@functools.lru_cache(maxsize=1)
def _sc_kernels():
    mesh = plsc.VectorSubcoreMesh(core_axis_name="c", subcore_axis_name="s",
                                  num_cores=NC, num_subcores=NS)
    sc_params = pltpu.CompilerParams(needs_layout_passes=False)
    combine = pl.kernel(
        _combine_body,
        out_type=jax.ShapeDtypeStruct((T, D), jnp.float32),
        mesh=mesh,
        scratch_types=[
            pltpu.VMEM((TPW,), jnp.int32),
            pltpu.VMEM((TPW, D), jnp.float32),
            pltpu.SemaphoreType.DMA,
        ],
        compiler_params=sc_params,
    )
    return combine


def kernel(x, Wr, br, W1, b1, W2, b2):
    _combine = _sc_kernels()
    xf = x.reshape(T, D)
    yg, slot2, counts2, scal = _ffn(xf, Wr, br.reshape(1, E), W1, W1,
                                    b1.reshape(E, 1, DFF), W2, W2,
                                    b2.reshape(E, 1, D))
    out_flat = _combine(yg, slot2.reshape(T))
    out = out_flat.reshape(x.shape)
    counts = counts2.reshape(E).astype(jnp.int32)
    return out, scal[0, 0], scal[0, 1], counts
